# Initial kernel scaffold; baseline (speedup 1.0000x reference)
import jax
import jax.numpy as jnp
from jax.experimental import pallas as pl


def kernel(img, W_enc, b_enc, W_tok, b_tok, w_score, Wz, Uz, bz, Wr, Ur, br, Wh, Uh, bh, W_dec, b_dec):
    raise NotImplementedError("write your pallas kernel here")


# speedup vs baseline: 1.8436x; 1.8436x over previous
import jax
import jax.numpy as jnp
from jax.experimental import pallas as pl
from jax.experimental.pallas import tpu as pltpu

_HI = jax.lax.Precision.HIGHEST


def _front_kernel(nk_const, patches_ref, Wenc_ref, benc_ref, Wtok_ref,
                  btok_ref, ws_ref, Wcat_ref, xproj_ref, mask_ref, idx_ref):
    N, K = nk_const
    p = patches_ref[0]
    feat = jnp.dot(p, Wenc_ref[...], preferred_element_type=jnp.float32)
    feat = feat + benc_ref[...]
    tok = jnp.dot(feat, Wtok_ref[...], preferred_element_type=jnp.float32)
    tok = jax.nn.gelu(tok + btok_ref[...])

    u = jnp.dot(tok, ws_ref[...], preferred_element_type=jnp.float32)

    onesN = jnp.ones((N, 1), jnp.float32)
    Lj = jax.lax.dot_general(onesN, u, (((1,), (1,)), ((), ())),
                             precision=_HI)
    Li = jnp.broadcast_to(u, (N, N))
    jIota = jax.lax.broadcasted_iota(jnp.int32, (N, N), 1).astype(jnp.float32)
    iIota = jax.lax.broadcasted_iota(jnp.int32, (N, N), 0).astype(jnp.float32)
    beats = (Lj > Li) | ((Lj == Li) & (jIota < iIota))
    rank = jnp.sum(beats.astype(jnp.float32), axis=1, keepdims=True)

    mask_ref[0] = (rank < K).astype(jnp.float32)

    onesK = jnp.ones((K, 1), jnp.float32)
    rankRow = jax.lax.dot_general(onesK, rank, (((1,), (1,)), ((), ())),
                                  precision=_HI)
    rIota = jax.lax.broadcasted_iota(jnp.int32, (K, N), 0).astype(jnp.float32)
    Psel = (rankRow == rIota).astype(jnp.float32)

    colIota = jax.lax.broadcasted_iota(jnp.int32, (K, N), 1).astype(jnp.float32)
    idx_ref[0] = jnp.sum(Psel * colIota, axis=1, keepdims=True)

    sel = jnp.dot(Psel, tok, precision=_HI)
    xproj_ref[0] = jnp.dot(sel, Wcat_ref[...],
                           preferred_element_type=jnp.float32)


def _gru_kernel(c_const, x_ref, Uzr_ref, Uh_ref, bzr_ref, bh_ref,
                hs_ref, h_scr):
    C = c_const
    k = pl.program_id(0)

    @pl.when(k == 0)
    def _():
        h_scr[...] = jnp.zeros_like(h_scr)

    h = h_scr[...]
    x = x_ref[0]
    hu = jnp.dot(h, Uzr_ref[...], preferred_element_type=jnp.float32)
    hu = hu + bzr_ref[...]
    z = jax.nn.sigmoid(x[:, :C] + hu[:, :C])
    r = jax.nn.sigmoid(x[:, C:2 * C] + hu[:, C:])
    n = jnp.tanh(x[:, 2 * C:] +
                 jnp.dot(r * h, Uh_ref[...], preferred_element_type=jnp.float32) +
                 bh_ref[...])
    h = (1.0 - z) * h + z * n
    h_scr[...] = h
    hs_ref[0] = h


def _dec_kernel(nk_const, hs_ref, idx_ref, Wdec_ref, bdec_ref, out_ref):
    N, K = nk_const
    hsb = hs_ref[0]
    dec = jnp.dot(hsb, Wdec_ref[...], preferred_element_type=jnp.float32)
    idxf = idx_ref[0]
    onesN = jnp.ones((N, 1), jnp.float32)
    idxMat = jax.lax.dot_general(onesN, idxf, (((1,), (1,)), ((), ())),
                                 precision=_HI)
    iIota = jax.lax.broadcasted_iota(jnp.int32, (N, K), 0).astype(jnp.float32)
    Sc = (idxMat == iIota).astype(jnp.float32)
    out_ref[0] = jnp.dot(Sc, dec, precision=_HI) + bdec_ref[...]


def kernel(img, W_enc, b_enc, W_tok, b_tok, w_score, Wz, Uz, bz, Wr, Ur, br,
           Wh, Uh, bh, W_dec, b_dec, interpret=False):
    B, CIN, H, W = img.shape
    D, C = W_enc.shape
    K = 98
    P = 16
    N = (H // P) * (W // P)

    patches = img.reshape(B, CIN, H // P, P, W // P, P)
    patches = patches.transpose(0, 2, 4, 1, 3, 5).reshape(B, N, D)
    Wcat = jnp.concatenate([Wz, Wr, Wh], axis=1)
    Uzr = jnp.concatenate([Uz, Ur], axis=1)
    bzr = jnp.concatenate([bz, br]).reshape(1, 2 * C)
    bh2 = bh.reshape(1, C)
    benc = b_enc.reshape(1, C)
    btok = b_tok.reshape(1, C)
    ws = w_score.reshape(C, 1)
    bdec = b_dec.reshape(1, D)

    import functools
    f32 = jnp.float32

    xproj, maskc, idxc = pl.pallas_call(
        functools.partial(_front_kernel, (N, K)),
        grid=(B,),
        in_specs=[
            pl.BlockSpec((1, N, D), lambda b: (b, 0, 0)),
            pl.BlockSpec((D, C), lambda b: (0, 0)),
            pl.BlockSpec((1, C), lambda b: (0, 0)),
            pl.BlockSpec((C, C), lambda b: (0, 0)),
            pl.BlockSpec((1, C), lambda b: (0, 0)),
            pl.BlockSpec((C, 1), lambda b: (0, 0)),
            pl.BlockSpec((C, 3 * C), lambda b: (0, 0)),
        ],
        out_specs=[
            pl.BlockSpec((1, K, 3 * C), lambda b: (b, 0, 0)),
            pl.BlockSpec((1, N, 1), lambda b: (b, 0, 0)),
            pl.BlockSpec((1, K, 1), lambda b: (b, 0, 0)),
        ],
        out_shape=[
            jax.ShapeDtypeStruct((B, K, 3 * C), f32),
            jax.ShapeDtypeStruct((B, N, 1), f32),
            jax.ShapeDtypeStruct((B, K, 1), f32),
        ],
        interpret=interpret,
    )(patches, W_enc, benc, W_tok, btok, ws, Wcat)

    xproj_t = jnp.transpose(xproj, (1, 0, 2))
    hs = pl.pallas_call(
        functools.partial(_gru_kernel, C),
        grid=(K,),
        in_specs=[
            pl.BlockSpec((1, B, 3 * C), lambda k: (k, 0, 0)),
            pl.BlockSpec((C, 2 * C), lambda k: (0, 0)),
            pl.BlockSpec((C, C), lambda k: (0, 0)),
            pl.BlockSpec((1, 2 * C), lambda k: (0, 0)),
            pl.BlockSpec((1, C), lambda k: (0, 0)),
        ],
        out_specs=pl.BlockSpec((1, B, C), lambda k: (k, 0, 0)),
        out_shape=jax.ShapeDtypeStruct((K, B, C), f32),
        scratch_shapes=[pltpu.VMEM((B, C), f32)],
        interpret=interpret,
    )(xproj_t, Uzr, Uh, bzr, bh2)

    hs_t = jnp.transpose(hs, (1, 0, 2))
    out_patches = pl.pallas_call(
        functools.partial(_dec_kernel, (N, K)),
        grid=(B,),
        in_specs=[
            pl.BlockSpec((1, K, C), lambda b: (b, 0, 0)),
            pl.BlockSpec((1, K, 1), lambda b: (b, 0, 0)),
            pl.BlockSpec((D, D), lambda b: (0, 0)),
            pl.BlockSpec((1, D), lambda b: (0, 0)),
        ],
        out_specs=pl.BlockSpec((1, N, D), lambda b: (b, 0, 0)),
        out_shape=jax.ShapeDtypeStruct((B, N, D), f32),
        interpret=interpret,
    )(hs_t, idxc, W_dec, bdec)

    recon = out_patches.reshape(B, H // P, W // P, CIN, P, P)
    recon = recon.transpose(0, 3, 1, 4, 2, 5).reshape(B, CIN, H, W)
    mask = maskc.reshape(B, N)
    indices = idxc.reshape(B, K).astype(jnp.int32)
    return (recon, mask, indices)


# speedup vs baseline: 1.9090x; 1.0355x over previous
import jax
import jax.numpy as jnp
from jax.experimental import pallas as pl
from jax.experimental.pallas import tpu as pltpu

_HI = jax.lax.Precision.HIGHEST


def _front_kernel(nk_const, patches_ref, Wenc_ref, benc_ref, Wtok_ref,
                  btok_ref, ws_ref, Wcat_ref, xproj_ref, mask_ref, idx_ref):
    N, K = nk_const
    p = patches_ref[0]
    feat = jnp.dot(p, Wenc_ref[...], preferred_element_type=jnp.float32)
    feat = feat + benc_ref[...]
    tok = jnp.dot(feat, Wtok_ref[...], preferred_element_type=jnp.float32)
    tok = jax.nn.gelu(tok + btok_ref[...])

    u = jnp.dot(tok, ws_ref[...], preferred_element_type=jnp.float32)

    onesN = jnp.ones((N, 1), jnp.float32)
    Lj = jax.lax.dot_general(onesN, u, (((1,), (1,)), ((), ())),
                             precision=_HI)
    Li = jnp.broadcast_to(u, (N, N))
    jIota = jax.lax.broadcasted_iota(jnp.int32, (N, N), 1).astype(jnp.float32)
    iIota = jax.lax.broadcasted_iota(jnp.int32, (N, N), 0).astype(jnp.float32)
    beats = (Lj > Li) | ((Lj == Li) & (jIota < iIota))
    rank = jnp.sum(beats.astype(jnp.float32), axis=1, keepdims=True)

    mask_ref[0] = (rank < K).astype(jnp.float32)

    onesK = jnp.ones((K, 1), jnp.float32)
    rankRow = jax.lax.dot_general(onesK, rank, (((1,), (1,)), ((), ())),
                                  precision=_HI)
    rIota = jax.lax.broadcasted_iota(jnp.int32, (K, N), 0).astype(jnp.float32)
    Psel = (rankRow == rIota).astype(jnp.float32)

    colIota = jax.lax.broadcasted_iota(jnp.int32, (K, N), 1).astype(jnp.float32)
    idx_ref[0] = jnp.sum(Psel * colIota, axis=1, keepdims=True)

    sel = jnp.dot(Psel, tok, precision=_HI)
    xproj_ref[0] = jnp.dot(sel, Wcat_ref[...],
                           preferred_element_type=jnp.float32)


def _gru_kernel(ck_const, x_ref, Uzr_ref, Uh_ref, bzr_ref, bh_ref, hs_ref):
    C, K = ck_const
    B = x_ref.shape[0]
    Uzr = Uzr_ref[...]
    Uh = Uh_ref[...]
    bzr = bzr_ref[...]
    bh = bh_ref[...]

    def step(k, h):
        x = x_ref[:, k, :]
        hu = jnp.dot(h, Uzr, preferred_element_type=jnp.float32) + bzr
        z = jax.nn.sigmoid(x[:, :C] + hu[:, :C])
        r = jax.nn.sigmoid(x[:, C:2 * C] + hu[:, C:])
        n = jnp.tanh(x[:, 2 * C:] +
                     jnp.dot(r * h, Uh, preferred_element_type=jnp.float32) +
                     bh)
        h = (1.0 - z) * h + z * n
        hs_ref[:, k, :] = h
        return h

    jax.lax.fori_loop(0, K, step, jnp.zeros((B, C), jnp.float32))


def _dec_kernel(nk_const, hs_ref, idx_ref, Wdec_ref, bdec_ref, out_ref):
    N, K = nk_const
    hsb = hs_ref[0]
    dec = jnp.dot(hsb, Wdec_ref[...], preferred_element_type=jnp.float32)
    idxf = idx_ref[0]
    onesN = jnp.ones((N, 1), jnp.float32)
    idxMat = jax.lax.dot_general(onesN, idxf, (((1,), (1,)), ((), ())),
                                 precision=_HI)
    iIota = jax.lax.broadcasted_iota(jnp.int32, (N, K), 0).astype(jnp.float32)
    Sc = (idxMat == iIota).astype(jnp.float32)
    out_ref[0] = jnp.dot(Sc, dec, precision=_HI) + bdec_ref[...]


def kernel(img, W_enc, b_enc, W_tok, b_tok, w_score, Wz, Uz, bz, Wr, Ur, br,
           Wh, Uh, bh, W_dec, b_dec, interpret=False):
    B, CIN, H, W = img.shape
    D, C = W_enc.shape
    K = 98
    P = 16
    N = (H // P) * (W // P)

    patches = img.reshape(B, CIN, H // P, P, W // P, P)
    patches = patches.transpose(0, 2, 4, 1, 3, 5).reshape(B, N, D)
    Wcat = jnp.concatenate([Wz, Wr, Wh], axis=1)
    Uzr = jnp.concatenate([Uz, Ur], axis=1)
    bzr = jnp.concatenate([bz, br]).reshape(1, 2 * C)
    bh2 = bh.reshape(1, C)
    benc = b_enc.reshape(1, C)
    btok = b_tok.reshape(1, C)
    ws = w_score.reshape(C, 1)
    bdec = b_dec.reshape(1, D)

    import functools
    f32 = jnp.float32

    xproj, maskc, idxc = pl.pallas_call(
        functools.partial(_front_kernel, (N, K)),
        grid=(B,),
        in_specs=[
            pl.BlockSpec((1, N, D), lambda b: (b, 0, 0)),
            pl.BlockSpec((D, C), lambda b: (0, 0)),
            pl.BlockSpec((1, C), lambda b: (0, 0)),
            pl.BlockSpec((C, C), lambda b: (0, 0)),
            pl.BlockSpec((1, C), lambda b: (0, 0)),
            pl.BlockSpec((C, 1), lambda b: (0, 0)),
            pl.BlockSpec((C, 3 * C), lambda b: (0, 0)),
        ],
        out_specs=[
            pl.BlockSpec((1, K, 3 * C), lambda b: (b, 0, 0)),
            pl.BlockSpec((1, N, 1), lambda b: (b, 0, 0)),
            pl.BlockSpec((1, K, 1), lambda b: (b, 0, 0)),
        ],
        out_shape=[
            jax.ShapeDtypeStruct((B, K, 3 * C), f32),
            jax.ShapeDtypeStruct((B, N, 1), f32),
            jax.ShapeDtypeStruct((B, K, 1), f32),
        ],
        interpret=interpret,
    )(patches, W_enc, benc, W_tok, btok, ws, Wcat)

    hs_t = pl.pallas_call(
        functools.partial(_gru_kernel, (C, K)),
        in_specs=[
            pl.BlockSpec((B, K, 3 * C), lambda: (0, 0, 0)),
            pl.BlockSpec((C, 2 * C), lambda: (0, 0)),
            pl.BlockSpec((C, C), lambda: (0, 0)),
            pl.BlockSpec((1, 2 * C), lambda: (0, 0)),
            pl.BlockSpec((1, C), lambda: (0, 0)),
        ],
        out_specs=pl.BlockSpec((B, K, C), lambda: (0, 0, 0)),
        out_shape=jax.ShapeDtypeStruct((B, K, C), f32),
        interpret=interpret,
    )(xproj, Uzr, Uh, bzr, bh2)

    out_patches = pl.pallas_call(
        functools.partial(_dec_kernel, (N, K)),
        grid=(B,),
        in_specs=[
            pl.BlockSpec((1, K, C), lambda b: (b, 0, 0)),
            pl.BlockSpec((1, K, 1), lambda b: (b, 0, 0)),
            pl.BlockSpec((D, D), lambda b: (0, 0)),
            pl.BlockSpec((1, D), lambda b: (0, 0)),
        ],
        out_specs=pl.BlockSpec((1, N, D), lambda b: (b, 0, 0)),
        out_shape=jax.ShapeDtypeStruct((B, N, D), f32),
        interpret=interpret,
    )(hs_t, idxc, W_dec, bdec)

    recon = out_patches.reshape(B, H // P, W // P, CIN, P, P)
    recon = recon.transpose(0, 3, 1, 4, 2, 5).reshape(B, CIN, H, W)
    mask = maskc.reshape(B, N)
    indices = idxc.reshape(B, K).astype(jnp.int32)
    return (recon, mask, indices)


# speedup vs baseline: 1.9131x; 1.0022x over previous
import functools

import jax
import jax.numpy as jnp
from jax.experimental import pallas as pl
from jax.experimental.pallas import tpu as pltpu

_HI = jax.lax.Precision.HIGHEST


def _front_kernel(nk_const, patches_ref, Wenc_ref, benc_ref, Wtok_ref,
                  btok_ref, ws_ref, sel_ref, mask_ref, idx_ref):
    N, K = nk_const
    p = patches_ref[0]
    feat = jnp.dot(p, Wenc_ref[...], preferred_element_type=jnp.float32)
    feat = feat + benc_ref[...]
    tok = jnp.dot(feat, Wtok_ref[...], preferred_element_type=jnp.float32)
    tok = jax.nn.gelu(tok + btok_ref[...])

    u = jnp.dot(tok, ws_ref[...], preferred_element_type=jnp.float32)

    onesN = jnp.ones((N, 1), jnp.float32)
    Lj = jax.lax.dot_general(onesN, u, (((1,), (1,)), ((), ())),
                             precision=_HI)
    Li = jnp.broadcast_to(u, (N, N))
    jIota = jax.lax.broadcasted_iota(jnp.int32, (N, N), 1).astype(jnp.float32)
    iIota = jax.lax.broadcasted_iota(jnp.int32, (N, N), 0).astype(jnp.float32)
    beats = (Lj > Li) | ((Lj == Li) & (jIota < iIota))
    rank = jnp.sum(beats.astype(jnp.float32), axis=1, keepdims=True)

    mask_ref[0] = (rank < K).astype(jnp.float32)

    onesK = jnp.ones((K, 1), jnp.float32)
    rankRow = jax.lax.dot_general(onesK, rank, (((1,), (1,)), ((), ())),
                                  precision=_HI)
    rIota = jax.lax.broadcasted_iota(jnp.int32, (K, N), 0).astype(jnp.float32)
    Psel = (rankRow == rIota).astype(jnp.float32)

    colIota = jax.lax.broadcasted_iota(jnp.int32, (K, N), 1).astype(jnp.float32)
    idx_ref[0] = jnp.sum(Psel * colIota, axis=1, keepdims=True)

    sel_ref[0] = jnp.dot(Psel, tok, precision=_HI)


def _gru_kernel(ck_const, sel_ref, Wz_ref, Wr_ref, Wh_ref, Uz_ref, Ur_ref,
                Uh_ref, bz_ref, br_ref, bh_ref, Wdec_ref, dec_ref,
                xz_scr, xr_scr, xh_scr, hs_scr):
    C, K = ck_const
    B = sel_ref.shape[0]
    bf16 = jnp.bfloat16

    selb = sel_ref[...].reshape(B * K, C).astype(bf16)
    xz_scr[...] = (jnp.dot(selb, Wz_ref[...], preferred_element_type=jnp.float32)
                   + bz_ref[...]).reshape(B, K, C)
    xr_scr[...] = (jnp.dot(selb, Wr_ref[...], preferred_element_type=jnp.float32)
                   + br_ref[...]).reshape(B, K, C)
    xh_scr[...] = (jnp.dot(selb, Wh_ref[...], preferred_element_type=jnp.float32)
                   + bh_ref[...]).reshape(B, K, C)

    Uz = Uz_ref[...]
    Ur = Ur_ref[...]
    Uh = Uh_ref[...]

    def step(k, h):
        hb = h.astype(bf16)
        z = jax.nn.sigmoid(
            xz_scr[:, k, :] +
            jnp.dot(hb, Uz, preferred_element_type=jnp.float32))
        r = jax.nn.sigmoid(
            xr_scr[:, k, :] +
            jnp.dot(hb, Ur, preferred_element_type=jnp.float32))
        n = jnp.tanh(
            xh_scr[:, k, :] +
            jnp.dot((r * h).astype(bf16), Uh, preferred_element_type=jnp.float32))
        h = (1.0 - z) * h + z * n
        hs_scr[:, k, :] = h
        return h

    jax.lax.fori_loop(0, K, step, jnp.zeros((B, C), jnp.float32))

    hsb = hs_scr[...].reshape(B * K, C).astype(bf16)
    dec_ref[...] = jnp.dot(hsb, Wdec_ref[...],
                           preferred_element_type=jnp.float32).reshape(
                               B, K, Wdec_ref.shape[1])


def _scatter_kernel(nk_const, dec_ref, idx_ref, bdec_ref, out_ref):
    N, K = nk_const
    decb = dec_ref[0]
    idxf = idx_ref[0]
    onesN = jnp.ones((N, 1), jnp.float32)
    idxMat = jax.lax.dot_general(onesN, idxf, (((1,), (1,)), ((), ())),
                                 precision=_HI)
    iIota = jax.lax.broadcasted_iota(jnp.int32, (N, K), 0).astype(jnp.float32)
    Sc = (idxMat == iIota).astype(jnp.float32)
    out_ref[0] = jnp.dot(Sc, decb, precision=_HI) + bdec_ref[...]


def kernel(img, W_enc, b_enc, W_tok, b_tok, w_score, Wz, Uz, bz, Wr, Ur, br,
           Wh, Uh, bh, W_dec, b_dec, interpret=False):
    B, CIN, H, W = img.shape
    D, C = W_enc.shape
    K = 98
    P = 16
    N = (H // P) * (W // P)
    f32 = jnp.float32
    bf16 = jnp.bfloat16

    patches = img.reshape(B, CIN, H // P, P, W // P, P)
    patches = patches.transpose(0, 2, 4, 1, 3, 5).reshape(B, N, D)
    benc = b_enc.reshape(1, C)
    btok = b_tok.reshape(1, C)
    ws = w_score.reshape(C, 1)
    bz2 = bz.reshape(1, C)
    br2 = br.reshape(1, C)
    bh2 = bh.reshape(1, C)
    bdec = b_dec.reshape(1, D)
    Wzb, Wrb, Whb = Wz.astype(bf16), Wr.astype(bf16), Wh.astype(bf16)
    Uzb, Urb, Uhb = Uz.astype(bf16), Ur.astype(bf16), Uh.astype(bf16)
    Wdecb = W_dec.astype(bf16)

    sel, maskc, idxc = pl.pallas_call(
        functools.partial(_front_kernel, (N, K)),
        grid=(B,),
        in_specs=[
            pl.BlockSpec((1, N, D), lambda b: (b, 0, 0)),
            pl.BlockSpec((D, C), lambda b: (0, 0)),
            pl.BlockSpec((1, C), lambda b: (0, 0)),
            pl.BlockSpec((C, C), lambda b: (0, 0)),
            pl.BlockSpec((1, C), lambda b: (0, 0)),
            pl.BlockSpec((C, 1), lambda b: (0, 0)),
        ],
        out_specs=[
            pl.BlockSpec((1, K, C), lambda b: (b, 0, 0)),
            pl.BlockSpec((1, N, 1), lambda b: (b, 0, 0)),
            pl.BlockSpec((1, K, 1), lambda b: (b, 0, 0)),
        ],
        out_shape=[
            jax.ShapeDtypeStruct((B, K, C), f32),
            jax.ShapeDtypeStruct((B, N, 1), f32),
            jax.ShapeDtypeStruct((B, K, 1), f32),
        ],
        interpret=interpret,
    )(patches, W_enc, benc, W_tok, btok, ws)

    dec = pl.pallas_call(
        functools.partial(_gru_kernel, (C, K)),
        in_specs=[
            pl.BlockSpec((B, K, C), lambda: (0, 0, 0)),
            pl.BlockSpec((C, C), lambda: (0, 0)),
            pl.BlockSpec((C, C), lambda: (0, 0)),
            pl.BlockSpec((C, C), lambda: (0, 0)),
            pl.BlockSpec((C, C), lambda: (0, 0)),
            pl.BlockSpec((C, C), lambda: (0, 0)),
            pl.BlockSpec((C, C), lambda: (0, 0)),
            pl.BlockSpec((1, C), lambda: (0, 0)),
            pl.BlockSpec((1, C), lambda: (0, 0)),
            pl.BlockSpec((1, C), lambda: (0, 0)),
            pl.BlockSpec((C, D), lambda: (0, 0)),
        ],
        out_specs=pl.BlockSpec((B, K, D), lambda: (0, 0, 0)),
        out_shape=jax.ShapeDtypeStruct((B, K, D), f32),
        scratch_shapes=[
            pltpu.VMEM((B, K, C), f32),
            pltpu.VMEM((B, K, C), f32),
            pltpu.VMEM((B, K, C), f32),
            pltpu.VMEM((B, K, C), f32),
        ],
        interpret=interpret,
    )(sel, Wzb, Wrb, Whb, Uzb, Urb, Uhb, bz2, br2, bh2, Wdecb)

    out_patches = pl.pallas_call(
        functools.partial(_scatter_kernel, (N, K)),
        grid=(B,),
        in_specs=[
            pl.BlockSpec((1, K, D), lambda b: (b, 0, 0)),
            pl.BlockSpec((1, K, 1), lambda b: (b, 0, 0)),
            pl.BlockSpec((1, D), lambda b: (0, 0)),
        ],
        out_specs=pl.BlockSpec((1, N, D), lambda b: (b, 0, 0)),
        out_shape=jax.ShapeDtypeStruct((B, N, D), f32),
        interpret=interpret,
    )(dec, idxc, bdec)

    recon = out_patches.reshape(B, H // P, W // P, CIN, P, P)
    recon = recon.transpose(0, 3, 1, 4, 2, 5).reshape(B, CIN, H, W)
    mask = maskc.reshape(B, N)
    indices = idxc.reshape(B, K).astype(jnp.int32)
    return (recon, mask, indices)


# speedup vs baseline: 1.9177x; 1.0024x over previous
import functools

import jax
import jax.numpy as jnp
from jax.experimental import pallas as pl
from jax.experimental.pallas import tpu as pltpu

_HI = jax.lax.Precision.HIGHEST


def _front_kernel(nk_const, patches_ref, Wenc_ref, benc_ref, Wtok_ref,
                  btok_ref, ws_ref, sel_ref, mask_ref, idx_ref):
    N, K, KP = nk_const
    p = patches_ref[0]
    feat = jnp.dot(p, Wenc_ref[...], preferred_element_type=jnp.float32)
    feat = feat + benc_ref[...]
    tok = jnp.dot(feat, Wtok_ref[...], preferred_element_type=jnp.float32)
    tok = jax.nn.gelu(tok + btok_ref[...])

    u = jnp.dot(tok, ws_ref[...], preferred_element_type=jnp.float32)

    onesN = jnp.ones((N, 1), jnp.float32)
    Lj = jax.lax.dot_general(onesN, u, (((1,), (1,)), ((), ())),
                             precision=_HI)
    Li = jnp.broadcast_to(u, (N, N))
    jIota = jax.lax.broadcasted_iota(jnp.int32, (N, N), 1).astype(jnp.float32)
    iIota = jax.lax.broadcasted_iota(jnp.int32, (N, N), 0).astype(jnp.float32)
    beats = (Lj > Li) | ((Lj == Li) & (jIota < iIota))
    rank = jnp.sum(beats.astype(jnp.float32), axis=1, keepdims=True)

    mask_ref[0] = (rank < K).astype(jnp.float32)

    onesK = jnp.ones((KP, 1), jnp.float32)
    rankRow = jax.lax.dot_general(onesK, rank, (((1,), (1,)), ((), ())),
                                  precision=_HI)
    rIota = jax.lax.broadcasted_iota(jnp.int32, (KP, N), 0).astype(jnp.float32)
    Psel = (rankRow == rIota).astype(jnp.float32)

    colIota = jax.lax.broadcasted_iota(jnp.int32, (KP, N), 1).astype(jnp.float32)
    idxf = jnp.sum(Psel * colIota, axis=1, keepdims=True)
    rowIota = jax.lax.broadcasted_iota(jnp.int32, (KP, 1), 0).astype(jnp.float32)
    idx_ref[0] = jnp.where(rowIota < K, idxf, float(N))

    sel_ref[0] = jnp.dot(Psel, tok, precision=_HI)


def _gru_kernel(ck_const, sel_ref, Wz_ref, Wr_ref, Wh_ref, Uz_ref, Ur_ref,
                Uh_ref, bz_ref, br_ref, bh_ref, Wdec_ref, dec_ref,
                xz_scr, xr_scr, xh_scr, hs_scr):
    C, K, KP = ck_const
    B = sel_ref.shape[0]
    bf16 = jnp.bfloat16

    hs_scr[:, K:, :] = jnp.zeros((B, KP - K, C), jnp.float32)
    selb = sel_ref[...].reshape(B * KP, C).astype(bf16)
    xz_scr[...] = (jnp.dot(selb, Wz_ref[...], preferred_element_type=jnp.float32)
                   + bz_ref[...]).reshape(B, KP, C)
    xr_scr[...] = (jnp.dot(selb, Wr_ref[...], preferred_element_type=jnp.float32)
                   + br_ref[...]).reshape(B, KP, C)
    xh_scr[...] = (jnp.dot(selb, Wh_ref[...], preferred_element_type=jnp.float32)
                   + bh_ref[...]).reshape(B, KP, C)

    Uz = Uz_ref[...]
    Ur = Ur_ref[...]
    Uh = Uh_ref[...]

    def step(k, h):
        hb = h.astype(bf16)
        z = jax.nn.sigmoid(
            xz_scr[:, k, :] +
            jnp.dot(hb, Uz, preferred_element_type=jnp.float32))
        r = jax.nn.sigmoid(
            xr_scr[:, k, :] +
            jnp.dot(hb, Ur, preferred_element_type=jnp.float32))
        n = jnp.tanh(
            xh_scr[:, k, :] +
            jnp.dot((r * h).astype(bf16), Uh, preferred_element_type=jnp.float32))
        h = (1.0 - z) * h + z * n
        hs_scr[:, k, :] = h
        return h

    jax.lax.fori_loop(0, K, step, jnp.zeros((B, C), jnp.float32))

    hsb = hs_scr[...].reshape(B * KP, C).astype(bf16)
    dec_ref[...] = jnp.dot(hsb, Wdec_ref[...],
                           preferred_element_type=jnp.float32).reshape(
                               B, KP, Wdec_ref.shape[1])


def _scatter_kernel(nk_const, dec_ref, idx_ref, bdec_ref, out_ref):
    N, KP = nk_const
    decb = dec_ref[0]
    idxf = idx_ref[0]
    onesN = jnp.ones((N, 1), jnp.float32)
    idxMat = jax.lax.dot_general(onesN, idxf, (((1,), (1,)), ((), ())),
                                 precision=_HI)
    iIota = jax.lax.broadcasted_iota(jnp.int32, (N, KP), 0).astype(jnp.float32)
    Sc = (idxMat == iIota).astype(jnp.float32)
    out_ref[0] = jnp.dot(Sc, decb, precision=_HI) + bdec_ref[...]


def kernel(img, W_enc, b_enc, W_tok, b_tok, w_score, Wz, Uz, bz, Wr, Ur, br,
           Wh, Uh, bh, W_dec, b_dec, interpret=False):
    B, CIN, H, W = img.shape
    D, C = W_enc.shape
    K = 98
    KP = 104
    P = 16
    N = (H // P) * (W // P)
    f32 = jnp.float32
    bf16 = jnp.bfloat16

    patches = img.reshape(B, CIN, H // P, P, W // P, P)
    patches = patches.transpose(0, 2, 4, 1, 3, 5).reshape(B, N, D)
    benc = b_enc.reshape(1, C)
    btok = b_tok.reshape(1, C)
    ws = w_score.reshape(C, 1)
    bz2 = bz.reshape(1, C)
    br2 = br.reshape(1, C)
    bh2 = bh.reshape(1, C)
    bdec = b_dec.reshape(1, D)
    Wzb, Wrb, Whb = Wz.astype(bf16), Wr.astype(bf16), Wh.astype(bf16)
    Uzb, Urb, Uhb = Uz.astype(bf16), Ur.astype(bf16), Uh.astype(bf16)
    Wdecb = W_dec.astype(bf16)

    sel, maskc, idxc = pl.pallas_call(
        functools.partial(_front_kernel, (N, K, KP)),
        grid=(B,),
        in_specs=[
            pl.BlockSpec((1, N, D), lambda b: (b, 0, 0)),
            pl.BlockSpec((D, C), lambda b: (0, 0)),
            pl.BlockSpec((1, C), lambda b: (0, 0)),
            pl.BlockSpec((C, C), lambda b: (0, 0)),
            pl.BlockSpec((1, C), lambda b: (0, 0)),
            pl.BlockSpec((C, 1), lambda b: (0, 0)),
        ],
        out_specs=[
            pl.BlockSpec((1, KP, C), lambda b: (b, 0, 0)),
            pl.BlockSpec((1, N, 1), lambda b: (b, 0, 0)),
            pl.BlockSpec((1, KP, 1), lambda b: (b, 0, 0)),
        ],
        out_shape=[
            jax.ShapeDtypeStruct((B, KP, C), f32),
            jax.ShapeDtypeStruct((B, N, 1), f32),
            jax.ShapeDtypeStruct((B, KP, 1), f32),
        ],
        interpret=interpret,
    )(patches, W_enc, benc, W_tok, btok, ws)

    dec = pl.pallas_call(
        functools.partial(_gru_kernel, (C, K, KP)),
        in_specs=[
            pl.BlockSpec((B, KP, C), lambda: (0, 0, 0)),
            pl.BlockSpec((C, C), lambda: (0, 0)),
            pl.BlockSpec((C, C), lambda: (0, 0)),
            pl.BlockSpec((C, C), lambda: (0, 0)),
            pl.BlockSpec((C, C), lambda: (0, 0)),
            pl.BlockSpec((C, C), lambda: (0, 0)),
            pl.BlockSpec((C, C), lambda: (0, 0)),
            pl.BlockSpec((1, C), lambda: (0, 0)),
            pl.BlockSpec((1, C), lambda: (0, 0)),
            pl.BlockSpec((1, C), lambda: (0, 0)),
            pl.BlockSpec((C, D), lambda: (0, 0)),
        ],
        out_specs=pl.BlockSpec((B, KP, D), lambda: (0, 0, 0)),
        out_shape=jax.ShapeDtypeStruct((B, KP, D), f32),
        scratch_shapes=[
            pltpu.VMEM((B, KP, C), f32),
            pltpu.VMEM((B, KP, C), f32),
            pltpu.VMEM((B, KP, C), f32),
            pltpu.VMEM((B, KP, C), f32),
        ],
        interpret=interpret,
    )(sel, Wzb, Wrb, Whb, Uzb, Urb, Uhb, bz2, br2, bh2, Wdecb)

    out_patches = pl.pallas_call(
        functools.partial(_scatter_kernel, (N, KP)),
        grid=(B,),
        in_specs=[
            pl.BlockSpec((1, KP, D), lambda b: (b, 0, 0)),
            pl.BlockSpec((1, KP, 1), lambda b: (b, 0, 0)),
            pl.BlockSpec((1, D), lambda b: (0, 0)),
        ],
        out_specs=pl.BlockSpec((1, N, D), lambda b: (b, 0, 0)),
        out_shape=jax.ShapeDtypeStruct((B, N, D), f32),
        interpret=interpret,
    )(dec, idxc, bdec)

    recon = out_patches.reshape(B, H // P, W // P, CIN, P, P)
    recon = recon.transpose(0, 3, 1, 4, 2, 5).reshape(B, CIN, H, W)
    mask = maskc.reshape(B, N)
    indices = idxc.reshape(B, KP)[:, :K].astype(jnp.int32)
    return (recon, mask, indices)


# speedup vs baseline: 2.0466x; 1.0672x over previous
import functools

import jax
import jax.numpy as jnp
from jax.experimental import pallas as pl
from jax.experimental.pallas import tpu as pltpu

_HI = jax.lax.Precision.HIGHEST


def _front_kernel(nk_const, patches_ref, Wenc_ref, benc_ref, Wtok_ref,
                  btok_ref, ws_ref, sel_ref, mask_ref, idx_ref):
    N, K, KP = nk_const
    p = patches_ref[0]
    feat = jnp.dot(p, Wenc_ref[...], preferred_element_type=jnp.float32)
    feat = feat + benc_ref[...]
    tok = jnp.dot(feat.astype(jnp.bfloat16), Wtok_ref[...],
                  preferred_element_type=jnp.float32)
    tok = jax.nn.gelu(tok + btok_ref[...])

    u = jnp.dot(tok, ws_ref[...], preferred_element_type=jnp.float32)

    onesN = jnp.ones((N, 1), jnp.float32)
    Lj = jax.lax.dot_general(onesN, u, (((1,), (1,)), ((), ())),
                             precision=_HI)
    Li = jnp.broadcast_to(u, (N, N))
    jIota = jax.lax.broadcasted_iota(jnp.int32, (N, N), 1).astype(jnp.float32)
    iIota = jax.lax.broadcasted_iota(jnp.int32, (N, N), 0).astype(jnp.float32)
    beats = (Lj > Li) | ((Lj == Li) & (jIota < iIota))
    rank = jnp.sum(beats.astype(jnp.float32), axis=1, keepdims=True)

    mask_ref[0] = (rank < K).astype(jnp.float32)

    onesK = jnp.ones((KP, 1), jnp.float32)
    rankRow = jax.lax.dot_general(onesK, rank, (((1,), (1,)), ((), ())),
                                  precision=_HI)
    rIota = jax.lax.broadcasted_iota(jnp.int32, (KP, N), 0).astype(jnp.float32)
    Psel = (rankRow == rIota).astype(jnp.float32)

    colIota = jax.lax.broadcasted_iota(jnp.int32, (KP, N), 1).astype(jnp.float32)
    idxf = jnp.sum(Psel * colIota, axis=1, keepdims=True)
    rowIota = jax.lax.broadcasted_iota(jnp.int32, (KP, 1), 0).astype(jnp.float32)
    idx_ref[0] = jnp.where(rowIota < K, idxf, float(N))

    sel_ref[0] = jnp.dot(Psel, tok, precision=_HI)


def _gru_kernel(ck_const, sel_ref, Wz_ref, Wr_ref, Wh_ref, Uz_ref, Ur_ref,
                Uh_ref, bz_ref, br_ref, bh_ref, Wdec_ref, dec_ref,
                xz_scr, xr_scr, xh_scr, hs_scr):
    C, K, KP = ck_const
    B = sel_ref.shape[0]
    bf16 = jnp.bfloat16

    hs_scr[:, K:, :] = jnp.zeros((B, KP - K, C), jnp.float32)
    selb = sel_ref[...].reshape(B * KP, C).astype(bf16)
    xz_scr[...] = (jnp.dot(selb, Wz_ref[...], preferred_element_type=jnp.float32)
                   + bz_ref[...]).reshape(B, KP, C)
    xr_scr[...] = (jnp.dot(selb, Wr_ref[...], preferred_element_type=jnp.float32)
                   + br_ref[...]).reshape(B, KP, C)
    xh_scr[...] = (jnp.dot(selb, Wh_ref[...], preferred_element_type=jnp.float32)
                   + bh_ref[...]).reshape(B, KP, C)

    Uz = Uz_ref[...]
    Ur = Ur_ref[...]
    Uh = Uh_ref[...]

    def step(k, h):
        hb = h.astype(bf16)
        z = jax.nn.sigmoid(
            xz_scr[:, k, :] +
            jnp.dot(hb, Uz, preferred_element_type=jnp.float32))
        r = jax.nn.sigmoid(
            xr_scr[:, k, :] +
            jnp.dot(hb, Ur, preferred_element_type=jnp.float32))
        n = jnp.tanh(
            xh_scr[:, k, :] +
            jnp.dot((r * h).astype(bf16), Uh, preferred_element_type=jnp.float32))
        h = (1.0 - z) * h + z * n
        hs_scr[:, k, :] = h
        return h

    jax.lax.fori_loop(0, K, step, jnp.zeros((B, C), jnp.float32))

    hsb = hs_scr[...].reshape(B * KP, C).astype(bf16)
    dec_ref[...] = jnp.dot(hsb, Wdec_ref[...],
                           preferred_element_type=jnp.float32).reshape(
                               B, KP, Wdec_ref.shape[1])


def _scatter_kernel(nk_const, dec_ref, idx_ref, bdec_ref, out_ref):
    N, KP = nk_const
    decb = dec_ref[0]
    idxf = idx_ref[0]
    onesN = jnp.ones((N, 1), jnp.float32)
    idxMat = jax.lax.dot_general(onesN, idxf, (((1,), (1,)), ((), ())),
                                 precision=_HI)
    iIota = jax.lax.broadcasted_iota(jnp.int32, (N, KP), 0).astype(jnp.float32)
    Sc = (idxMat == iIota).astype(jnp.float32)
    out_ref[0] = jnp.dot(Sc, decb, precision=_HI) + bdec_ref[...]


def kernel(img, W_enc, b_enc, W_tok, b_tok, w_score, Wz, Uz, bz, Wr, Ur, br,
           Wh, Uh, bh, W_dec, b_dec, interpret=False):
    B, CIN, H, W = img.shape
    D, C = W_enc.shape
    K = 98
    KP = 104
    P = 16
    N = (H // P) * (W // P)
    f32 = jnp.float32
    bf16 = jnp.bfloat16

    patches = img.astype(bf16).reshape(B, CIN, H // P, P, W // P, P)
    patches = patches.transpose(0, 2, 4, 1, 3, 5).reshape(B, N, D)
    benc = b_enc.reshape(1, C)
    btok = b_tok.reshape(1, C)
    ws = w_score.reshape(C, 1)
    bz2 = bz.reshape(1, C)
    br2 = br.reshape(1, C)
    bh2 = bh.reshape(1, C)
    bdec = b_dec.reshape(1, D)
    Wzb, Wrb, Whb = Wz.astype(bf16), Wr.astype(bf16), Wh.astype(bf16)
    Uzb, Urb, Uhb = Uz.astype(bf16), Ur.astype(bf16), Uh.astype(bf16)
    Wdecb = W_dec.astype(bf16)
    Wencb = W_enc.astype(bf16)
    Wtokb = W_tok.astype(bf16)

    sel, maskc, idxc = pl.pallas_call(
        functools.partial(_front_kernel, (N, K, KP)),
        grid=(B,),
        in_specs=[
            pl.BlockSpec((1, N, D), lambda b: (b, 0, 0)),
            pl.BlockSpec((D, C), lambda b: (0, 0)),
            pl.BlockSpec((1, C), lambda b: (0, 0)),
            pl.BlockSpec((C, C), lambda b: (0, 0)),
            pl.BlockSpec((1, C), lambda b: (0, 0)),
            pl.BlockSpec((C, 1), lambda b: (0, 0)),
        ],
        out_specs=[
            pl.BlockSpec((1, KP, C), lambda b: (b, 0, 0)),
            pl.BlockSpec((1, N, 1), lambda b: (b, 0, 0)),
            pl.BlockSpec((1, KP, 1), lambda b: (b, 0, 0)),
        ],
        out_shape=[
            jax.ShapeDtypeStruct((B, KP, C), f32),
            jax.ShapeDtypeStruct((B, N, 1), f32),
            jax.ShapeDtypeStruct((B, KP, 1), f32),
        ],
        interpret=interpret,
    )(patches, Wencb, benc, Wtokb, btok, ws)

    dec = pl.pallas_call(
        functools.partial(_gru_kernel, (C, K, KP)),
        in_specs=[
            pl.BlockSpec((B, KP, C), lambda: (0, 0, 0)),
            pl.BlockSpec((C, C), lambda: (0, 0)),
            pl.BlockSpec((C, C), lambda: (0, 0)),
            pl.BlockSpec((C, C), lambda: (0, 0)),
            pl.BlockSpec((C, C), lambda: (0, 0)),
            pl.BlockSpec((C, C), lambda: (0, 0)),
            pl.BlockSpec((C, C), lambda: (0, 0)),
            pl.BlockSpec((1, C), lambda: (0, 0)),
            pl.BlockSpec((1, C), lambda: (0, 0)),
            pl.BlockSpec((1, C), lambda: (0, 0)),
            pl.BlockSpec((C, D), lambda: (0, 0)),
        ],
        out_specs=pl.BlockSpec((B, KP, D), lambda: (0, 0, 0)),
        out_shape=jax.ShapeDtypeStruct((B, KP, D), f32),
        scratch_shapes=[
            pltpu.VMEM((B, KP, C), f32),
            pltpu.VMEM((B, KP, C), f32),
            pltpu.VMEM((B, KP, C), f32),
            pltpu.VMEM((B, KP, C), f32),
        ],
        interpret=interpret,
    )(sel, Wzb, Wrb, Whb, Uzb, Urb, Uhb, bz2, br2, bh2, Wdecb)

    out_patches = pl.pallas_call(
        functools.partial(_scatter_kernel, (N, KP)),
        grid=(B,),
        in_specs=[
            pl.BlockSpec((1, KP, D), lambda b: (b, 0, 0)),
            pl.BlockSpec((1, KP, 1), lambda b: (b, 0, 0)),
            pl.BlockSpec((1, D), lambda b: (0, 0)),
        ],
        out_specs=pl.BlockSpec((1, N, D), lambda b: (b, 0, 0)),
        out_shape=jax.ShapeDtypeStruct((B, N, D), f32),
        interpret=interpret,
    )(dec, idxc, bdec)

    recon = out_patches.reshape(B, H // P, W // P, CIN, P, P)
    recon = recon.transpose(0, 3, 1, 4, 2, 5).reshape(B, CIN, H, W)
    mask = maskc.reshape(B, N)
    indices = idxc.reshape(B, KP)[:, :K].astype(jnp.int32)
    return (recon, mask, indices)


# speedup vs baseline: 3.7469x; 1.8307x over previous
import functools

import jax
import jax.numpy as jnp
from jax.experimental import pallas as pl
from jax.experimental.pallas import tpu as pltpu

_HI = jax.lax.Precision.HIGHEST


def _front_kernel(nk_const, img_ref, Wenc_ref, benc_ref, Wtok_ref,
                  btok_ref, ws_ref, sel_ref, mask_ref, idx_ref):
    N, K, KP = nk_const
    p = pltpu.einshape("c(iq)(jw)->(ij)(cqw)", img_ref[0], q=16, w=16)
    feat = jnp.dot(p, Wenc_ref[...], preferred_element_type=jnp.float32)
    feat = feat + benc_ref[...]
    tok = jnp.dot(feat.astype(jnp.bfloat16), Wtok_ref[...],
                  preferred_element_type=jnp.float32)
    tok = jax.nn.gelu(tok + btok_ref[...])

    u = jnp.dot(tok, ws_ref[...], preferred_element_type=jnp.float32)

    onesN = jnp.ones((N, 1), jnp.float32)
    Lj = jax.lax.dot_general(onesN, u, (((1,), (1,)), ((), ())),
                             precision=_HI)
    Li = jnp.broadcast_to(u, (N, N))
    jIota = jax.lax.broadcasted_iota(jnp.int32, (N, N), 1).astype(jnp.float32)
    iIota = jax.lax.broadcasted_iota(jnp.int32, (N, N), 0).astype(jnp.float32)
    beats = (Lj > Li) | ((Lj == Li) & (jIota < iIota))
    rank = jnp.sum(beats.astype(jnp.float32), axis=1, keepdims=True)

    mask_ref[0] = (rank < K).astype(jnp.float32)

    onesK = jnp.ones((KP, 1), jnp.float32)
    rankRow = jax.lax.dot_general(onesK, rank, (((1,), (1,)), ((), ())),
                                  precision=_HI)
    rIota = jax.lax.broadcasted_iota(jnp.int32, (KP, N), 0).astype(jnp.float32)
    Psel = (rankRow == rIota).astype(jnp.float32)

    colIota = jax.lax.broadcasted_iota(jnp.int32, (KP, N), 1).astype(jnp.float32)
    idxf = jnp.sum(Psel * colIota, axis=1, keepdims=True)
    rowIota = jax.lax.broadcasted_iota(jnp.int32, (KP, 1), 0).astype(jnp.float32)
    idx_ref[0] = jnp.where(rowIota < K, idxf, float(N))

    sel_ref[0] = jnp.dot(Psel, tok, precision=_HI)


def _gru_kernel(ck_const, sel_ref, Wz_ref, Wr_ref, Wh_ref, Uz_ref, Ur_ref,
                Uh_ref, bz_ref, br_ref, bh_ref, Wdec_ref, dec_ref,
                xz_scr, xr_scr, xh_scr, hs_scr):
    C, K, KP = ck_const
    B = sel_ref.shape[0]
    bf16 = jnp.bfloat16

    hs_scr[:, K:, :] = jnp.zeros((B, KP - K, C), jnp.float32)
    selb = sel_ref[...].reshape(B * KP, C).astype(bf16)
    xz_scr[...] = (jnp.dot(selb, Wz_ref[...], preferred_element_type=jnp.float32)
                   + bz_ref[...]).reshape(B, KP, C)
    xr_scr[...] = (jnp.dot(selb, Wr_ref[...], preferred_element_type=jnp.float32)
                   + br_ref[...]).reshape(B, KP, C)
    xh_scr[...] = (jnp.dot(selb, Wh_ref[...], preferred_element_type=jnp.float32)
                   + bh_ref[...]).reshape(B, KP, C)

    Uz = Uz_ref[...]
    Ur = Ur_ref[...]
    Uh = Uh_ref[...]

    def step(k, h):
        hb = h.astype(bf16)
        z = jax.nn.sigmoid(
            xz_scr[:, k, :] +
            jnp.dot(hb, Uz, preferred_element_type=jnp.float32))
        r = jax.nn.sigmoid(
            xr_scr[:, k, :] +
            jnp.dot(hb, Ur, preferred_element_type=jnp.float32))
        n = jnp.tanh(
            xh_scr[:, k, :] +
            jnp.dot((r * h).astype(bf16), Uh, preferred_element_type=jnp.float32))
        h = (1.0 - z) * h + z * n
        hs_scr[:, k, :] = h
        return h

    jax.lax.fori_loop(0, K, step, jnp.zeros((B, C), jnp.float32))

    hsb = hs_scr[...].reshape(B * KP, C).astype(bf16)
    dec_ref[...] = jnp.dot(hsb, Wdec_ref[...],
                           preferred_element_type=jnp.float32).reshape(
                               B, KP, Wdec_ref.shape[1])


def _scatter_kernel(nk_const, dec_ref, idx_ref, bdec_ref, out_ref):
    N, KP = nk_const
    decb = dec_ref[0]
    idxf = idx_ref[0]
    onesN = jnp.ones((N, 1), jnp.float32)
    idxMat = jax.lax.dot_general(onesN, idxf, (((1,), (1,)), ((), ())),
                                 precision=_HI)
    iIota = jax.lax.broadcasted_iota(jnp.int32, (N, KP), 0).astype(jnp.float32)
    Sc = (idxMat == iIota).astype(jnp.float32)
    y = jnp.dot(Sc, decb, precision=_HI) + bdec_ref[...]
    out_ref[0] = pltpu.einshape("(ij)(cqw)->c(iq)(jw)", y, i=14, c=3, q=16)


def kernel(img, W_enc, b_enc, W_tok, b_tok, w_score, Wz, Uz, bz, Wr, Ur, br,
           Wh, Uh, bh, W_dec, b_dec, interpret=False):
    B, CIN, H, W = img.shape
    D, C = W_enc.shape
    K = 98
    KP = 104
    P = 16
    N = (H // P) * (W // P)
    f32 = jnp.float32
    bf16 = jnp.bfloat16

    imgb = img.astype(bf16)
    benc = b_enc.reshape(1, C)
    btok = b_tok.reshape(1, C)
    ws = w_score.reshape(C, 1)
    bz2 = bz.reshape(1, C)
    br2 = br.reshape(1, C)
    bh2 = bh.reshape(1, C)
    bdec = b_dec.reshape(1, D)
    Wzb, Wrb, Whb = Wz.astype(bf16), Wr.astype(bf16), Wh.astype(bf16)
    Uzb, Urb, Uhb = Uz.astype(bf16), Ur.astype(bf16), Uh.astype(bf16)
    Wdecb = W_dec.astype(bf16)
    Wencb = W_enc.astype(bf16)
    Wtokb = W_tok.astype(bf16)

    sel, maskc, idxc = pl.pallas_call(
        functools.partial(_front_kernel, (N, K, KP)),
        grid=(B,),
        in_specs=[
            pl.BlockSpec((1, CIN, H, W), lambda b: (b, 0, 0, 0)),
            pl.BlockSpec((D, C), lambda b: (0, 0)),
            pl.BlockSpec((1, C), lambda b: (0, 0)),
            pl.BlockSpec((C, C), lambda b: (0, 0)),
            pl.BlockSpec((1, C), lambda b: (0, 0)),
            pl.BlockSpec((C, 1), lambda b: (0, 0)),
        ],
        out_specs=[
            pl.BlockSpec((1, KP, C), lambda b: (b, 0, 0)),
            pl.BlockSpec((1, N, 1), lambda b: (b, 0, 0)),
            pl.BlockSpec((1, KP, 1), lambda b: (b, 0, 0)),
        ],
        out_shape=[
            jax.ShapeDtypeStruct((B, KP, C), f32),
            jax.ShapeDtypeStruct((B, N, 1), f32),
            jax.ShapeDtypeStruct((B, KP, 1), f32),
        ],
        interpret=interpret,
    )(imgb, Wencb, benc, Wtokb, btok, ws)

    dec = pl.pallas_call(
        functools.partial(_gru_kernel, (C, K, KP)),
        in_specs=[
            pl.BlockSpec((B, KP, C), lambda: (0, 0, 0)),
            pl.BlockSpec((C, C), lambda: (0, 0)),
            pl.BlockSpec((C, C), lambda: (0, 0)),
            pl.BlockSpec((C, C), lambda: (0, 0)),
            pl.BlockSpec((C, C), lambda: (0, 0)),
            pl.BlockSpec((C, C), lambda: (0, 0)),
            pl.BlockSpec((C, C), lambda: (0, 0)),
            pl.BlockSpec((1, C), lambda: (0, 0)),
            pl.BlockSpec((1, C), lambda: (0, 0)),
            pl.BlockSpec((1, C), lambda: (0, 0)),
            pl.BlockSpec((C, D), lambda: (0, 0)),
        ],
        out_specs=pl.BlockSpec((B, KP, D), lambda: (0, 0, 0)),
        out_shape=jax.ShapeDtypeStruct((B, KP, D), f32),
        scratch_shapes=[
            pltpu.VMEM((B, KP, C), f32),
            pltpu.VMEM((B, KP, C), f32),
            pltpu.VMEM((B, KP, C), f32),
            pltpu.VMEM((B, KP, C), f32),
        ],
        interpret=interpret,
    )(sel, Wzb, Wrb, Whb, Uzb, Urb, Uhb, bz2, br2, bh2, Wdecb)

    out_patches = pl.pallas_call(
        functools.partial(_scatter_kernel, (N, KP)),
        grid=(B,),
        in_specs=[
            pl.BlockSpec((1, KP, D), lambda b: (b, 0, 0)),
            pl.BlockSpec((1, KP, 1), lambda b: (b, 0, 0)),
            pl.BlockSpec((1, D), lambda b: (0, 0)),
        ],
        out_specs=pl.BlockSpec((1, CIN, H, W), lambda b: (b, 0, 0, 0)),
        out_shape=jax.ShapeDtypeStruct((B, CIN, H, W), f32),
        interpret=interpret,
    )(dec, idxc, bdec)

    recon = out_patches
    mask = maskc.reshape(B, N)
    indices = idxc.reshape(B, KP)[:, :K].astype(jnp.int32)
    return (recon, mask, indices)


# speedup vs baseline: 4.1043x; 1.0954x over previous
import functools

import jax
import jax.numpy as jnp
from jax.experimental import pallas as pl
from jax.experimental.pallas import tpu as pltpu

_HI = jax.lax.Precision.HIGHEST


def _front_kernel(nk_const, img_ref, Wenc_ref, benc_ref, Wtok_ref,
                  btok_ref, ws_ref, sel_ref, mask_ref, idx_ref):
    N, K, KP = nk_const
    for s in range(img_ref.shape[0]):
        _front_one(N, K, KP, s, img_ref, Wenc_ref, benc_ref, Wtok_ref,
                   btok_ref, ws_ref, sel_ref, mask_ref, idx_ref)


def _front_one(N, K, KP, s, img_ref, Wenc_ref, benc_ref, Wtok_ref,
               btok_ref, ws_ref, sel_ref, mask_ref, idx_ref):
    p = pltpu.einshape("c(iq)(jw)->(ij)(cqw)", img_ref[s], q=16, w=16)
    feat = jnp.dot(p, Wenc_ref[...], preferred_element_type=jnp.float32)
    feat = feat + benc_ref[...]
    tok = jnp.dot(feat.astype(jnp.bfloat16), Wtok_ref[...],
                  preferred_element_type=jnp.float32)
    tok = jax.nn.gelu(tok + btok_ref[...])

    u = jnp.dot(tok, ws_ref[...], preferred_element_type=jnp.float32)

    onesN = jnp.ones((N, 1), jnp.float32)
    Lj = jax.lax.dot_general(onesN, u, (((1,), (1,)), ((), ())),
                             precision=_HI)
    Li = jnp.broadcast_to(u, (N, N))
    jIota = jax.lax.broadcasted_iota(jnp.int32, (N, N), 1).astype(jnp.float32)
    iIota = jax.lax.broadcasted_iota(jnp.int32, (N, N), 0).astype(jnp.float32)
    beats = (Lj > Li) | ((Lj == Li) & (jIota < iIota))
    rank = jnp.sum(beats.astype(jnp.float32), axis=1, keepdims=True)

    mask_ref[s] = (rank < K).astype(jnp.float32)

    onesK = jnp.ones((KP, 1), jnp.float32)
    rankRow = jax.lax.dot_general(onesK, rank, (((1,), (1,)), ((), ())))
    rIota = jax.lax.broadcasted_iota(jnp.int32, (KP, N), 0).astype(jnp.float32)
    Psel = (rankRow == rIota).astype(jnp.float32)

    colIota = jax.lax.broadcasted_iota(jnp.int32, (KP, N), 1).astype(jnp.float32)
    idxf = jnp.sum(Psel * colIota, axis=1, keepdims=True)
    rowIota = jax.lax.broadcasted_iota(jnp.int32, (KP, 1), 0).astype(jnp.float32)
    idx_ref[s] = jnp.where(rowIota < K, idxf, float(N))

    sel_ref[s] = jnp.dot(Psel, tok, precision=_HI)


def _gru_kernel(ck_const, sel_ref, Wz_ref, Wr_ref, Wh_ref, Uzr_ref,
                Uh_ref, bz_ref, br_ref, bh_ref, Wdec_ref, dec_ref,
                xz_scr, xr_scr, xh_scr, hs_scr):
    C, K, KP = ck_const
    B = sel_ref.shape[0]
    bf16 = jnp.bfloat16

    hs_scr[:, K:, :] = jnp.zeros((B, KP - K, C), jnp.float32)
    selb = sel_ref[...].reshape(B * KP, C).astype(bf16)
    xz_scr[...] = (jnp.dot(selb, Wz_ref[...], preferred_element_type=jnp.float32)
                   + bz_ref[...]).reshape(B, KP, C)
    xr_scr[...] = (jnp.dot(selb, Wr_ref[...], preferred_element_type=jnp.float32)
                   + br_ref[...]).reshape(B, KP, C)
    xh_scr[...] = (jnp.dot(selb, Wh_ref[...], preferred_element_type=jnp.float32)
                   + bh_ref[...]).reshape(B, KP, C)

    Uzr = Uzr_ref[...]
    Uh = Uh_ref[...]

    def one(k, h):
        hb = h.astype(bf16)
        hu = jnp.dot(hb, Uzr, preferred_element_type=jnp.float32)
        z = jax.nn.sigmoid(xz_scr[:, k, :] + hu[:, :C])
        r = jax.nn.sigmoid(xr_scr[:, k, :] + hu[:, C:])
        n = jnp.tanh(
            xh_scr[:, k, :] +
            jnp.dot((r * h).astype(bf16), Uh, preferred_element_type=jnp.float32))
        h = (1.0 - z) * h + z * n
        hs_scr[:, k, :] = h
        return h

    def step2(i, h):
        h = one(2 * i, h)
        return one(2 * i + 1, h)

    jax.lax.fori_loop(0, K // 2, step2, jnp.zeros((B, C), jnp.float32))

    hsb = hs_scr[...].reshape(B * KP, C).astype(bf16)
    dec_ref[...] = jnp.dot(hsb, Wdec_ref[...],
                           preferred_element_type=jnp.float32).reshape(
                               B, KP, Wdec_ref.shape[1])


def _scatter_kernel(nk_const, dec_ref, idx_ref, bdec_ref, out_ref):
    N, KP = nk_const
    for s in range(dec_ref.shape[0]):
        decb = dec_ref[s]
        idxf = idx_ref[s]
        onesN = jnp.ones((N, 1), jnp.float32)
        idxMat = jax.lax.dot_general(onesN, idxf, (((1,), (1,)), ((), ())))
        iIota = jax.lax.broadcasted_iota(jnp.int32, (N, KP), 0).astype(jnp.float32)
        Sc = (idxMat == iIota).astype(jnp.float32)
        y = jnp.dot(Sc, decb, precision=_HI) + bdec_ref[...]
        out_ref[s] = pltpu.einshape("(ij)(cqw)->c(iq)(jw)", y, i=14, c=3, q=16)


def kernel(img, W_enc, b_enc, W_tok, b_tok, w_score, Wz, Uz, bz, Wr, Ur, br,
           Wh, Uh, bh, W_dec, b_dec, interpret=False):
    B, CIN, H, W = img.shape
    D, C = W_enc.shape
    K = 98
    KP = 104
    P = 16
    N = (H // P) * (W // P)
    f32 = jnp.float32
    bf16 = jnp.bfloat16

    imgb = img.astype(bf16)
    benc = b_enc.reshape(1, C)
    btok = b_tok.reshape(1, C)
    ws = w_score.reshape(C, 1)
    bz2 = bz.reshape(1, C)
    br2 = br.reshape(1, C)
    bh2 = bh.reshape(1, C)
    bdec = b_dec.reshape(1, D)
    Wzb, Wrb, Whb = Wz.astype(bf16), Wr.astype(bf16), Wh.astype(bf16)
    Uzrb = jnp.concatenate([Uz, Ur], axis=1).astype(bf16)
    Uhb = Uh.astype(bf16)
    Wdecb = W_dec.astype(bf16)
    Wencb = W_enc.astype(bf16)
    Wtokb = W_tok.astype(bf16)

    sel, maskc, idxc = pl.pallas_call(
        functools.partial(_front_kernel, (N, K, KP)),
        grid=(B // 2,),
        in_specs=[
            pl.BlockSpec((2, CIN, H, W), lambda b: (b, 0, 0, 0)),
            pl.BlockSpec((D, C), lambda b: (0, 0)),
            pl.BlockSpec((1, C), lambda b: (0, 0)),
            pl.BlockSpec((C, C), lambda b: (0, 0)),
            pl.BlockSpec((1, C), lambda b: (0, 0)),
            pl.BlockSpec((C, 1), lambda b: (0, 0)),
        ],
        out_specs=[
            pl.BlockSpec((2, KP, C), lambda b: (b, 0, 0)),
            pl.BlockSpec((2, N, 1), lambda b: (b, 0, 0)),
            pl.BlockSpec((2, KP, 1), lambda b: (b, 0, 0)),
        ],
        out_shape=[
            jax.ShapeDtypeStruct((B, KP, C), f32),
            jax.ShapeDtypeStruct((B, N, 1), f32),
            jax.ShapeDtypeStruct((B, KP, 1), f32),
        ],
        interpret=interpret,
    )(imgb, Wencb, benc, Wtokb, btok, ws)

    dec = pl.pallas_call(
        functools.partial(_gru_kernel, (C, K, KP)),
        in_specs=[
            pl.BlockSpec((B, KP, C), lambda: (0, 0, 0)),
            pl.BlockSpec((C, C), lambda: (0, 0)),
            pl.BlockSpec((C, C), lambda: (0, 0)),
            pl.BlockSpec((C, C), lambda: (0, 0)),
            pl.BlockSpec((C, 2 * C), lambda: (0, 0)),
            pl.BlockSpec((C, C), lambda: (0, 0)),
            pl.BlockSpec((1, C), lambda: (0, 0)),
            pl.BlockSpec((1, C), lambda: (0, 0)),
            pl.BlockSpec((1, C), lambda: (0, 0)),
            pl.BlockSpec((C, D), lambda: (0, 0)),
        ],
        out_specs=pl.BlockSpec((B, KP, D), lambda: (0, 0, 0)),
        out_shape=jax.ShapeDtypeStruct((B, KP, D), f32),
        scratch_shapes=[
            pltpu.VMEM((B, KP, C), f32),
            pltpu.VMEM((B, KP, C), f32),
            pltpu.VMEM((B, KP, C), f32),
            pltpu.VMEM((B, KP, C), f32),
        ],
        interpret=interpret,
    )(sel, Wzb, Wrb, Whb, Uzrb, Uhb, bz2, br2, bh2, Wdecb)

    out_patches = pl.pallas_call(
        functools.partial(_scatter_kernel, (N, KP)),
        grid=(B // 2,),
        in_specs=[
            pl.BlockSpec((2, KP, D), lambda b: (b, 0, 0)),
            pl.BlockSpec((2, KP, 1), lambda b: (b, 0, 0)),
            pl.BlockSpec((1, D), lambda b: (0, 0)),
        ],
        out_specs=pl.BlockSpec((2, CIN, H, W), lambda b: (b, 0, 0, 0)),
        out_shape=jax.ShapeDtypeStruct((B, CIN, H, W), f32),
        interpret=interpret,
    )(dec, idxc, bdec)

    recon = out_patches
    mask = maskc.reshape(B, N)
    indices = idxc.reshape(B, KP)[:, :K].astype(jnp.int32)
    return (recon, mask, indices)
